# baseline (speedup 1.0000x reference)
import functools
import jax
import jax.numpy as jnp
from jax import lax
from jax.experimental import pallas as pl
from jax.experimental.pallas import tpu as pltpu
from jax.experimental.pallas import tpu_sc as plsc

EMBED = 64
PAD = 128
SEQ = 200
CHUNKS = ((0, 128), (128, 72))
NUM_WORKERS = 32
NUM_CHUNKS = 4
TC_ROWS = 8


def _sc_gather(word_padded, flat_idx, batch):
    num_idx = batch * SEQ
    rows_per_w = batch // NUM_WORKERS
    idx_per_w = rows_per_w * SEQ
    n_chunks = 2 * rows_per_w

    mesh = plsc.VectorSubcoreMesh(core_axis_name="c", subcore_axis_name="s")

    @functools.partial(
        pl.kernel,
        out_type=jax.ShapeDtypeStruct((num_idx, PAD), jnp.float32),
        mesh=mesh,
        scratch_types=[
            pltpu.VMEM((idx_per_w,), jnp.int32),
            pltpu.VMEM((4, CHUNKS[0][1], PAD), jnp.float32),
            pltpu.SemaphoreType.DMA,
            pltpu.SemaphoreType.DMA,
            pltpu.SemaphoreType.DMA,
            pltpu.SemaphoreType.DMA,
            pltpu.SemaphoreType.DMA,
            pltpu.SemaphoreType.DMA,
            pltpu.SemaphoreType.DMA,
            pltpu.SemaphoreType.DMA,
        ],
    )
    def sc_kernel(word_hbm, idx_hbm, out_hbm, idx_v, rows_v,
                  g0, g1, g2, g3, o0, o1, o2, o3):
        wid = lax.axis_index("s") * 2 + lax.axis_index("c")
        idx_base = pl.multiple_of(wid * idx_per_w, idx_per_w)
        pltpu.sync_copy(idx_hbm.at[pl.ds(idx_base, idx_per_w)], idx_v)

        gsems = (g0, g1, g2, g3)
        osems = (o0, o1, o2, o3)

        def gather(t, h, b):
            start, size = CHUNKS[h]
            return pltpu.make_async_copy(
                word_hbm.at[idx_v.at[pl.ds(t * SEQ + start, size)]],
                rows_v.at[b, pl.ds(0, size)], gsems[b],
            )

        def out_copy(t, h, b):
            start, size = CHUNKS[h]
            out_base = pl.multiple_of(idx_base + t * SEQ + start, 8)
            return pltpu.make_async_copy(
                rows_v.at[b, pl.ds(0, size)],
                out_hbm.at[pl.ds(out_base, size)], osems[b],
            )

        def do_chunk(t, h, b, wait_out, issue_next):
            gather(t, h, b).wait()
            out_copy(t, h, b).start()
            if wait_out:
                out_copy(t - 1, h, (b + 2) % 4).wait()
            if issue_next:
                gather(t + 1, h, (b + 2) % 4).start()

        gather(0, 0, 0).start()
        gather(0, 1, 1).start()
        do_chunk(0, 0, 0, wait_out=False, issue_next=True)
        do_chunk(0, 1, 1, wait_out=False, issue_next=True)

        @pl.loop(0, (rows_per_w - 2) // 2)
        def _(u):
            for k in range(4):
                t = 1 + 2 * u + k // 2
                do_chunk(t, k % 2, (2 + k) % 4, wait_out=True,
                         issue_next=True)

        do_chunk(rows_per_w - 1, 0, (2 * rows_per_w - 2) % 4, wait_out=True,
                 issue_next=False)
        do_chunk(rows_per_w - 1, 1, (2 * rows_per_w - 1) % 4, wait_out=True,
                 issue_next=False)
        out_copy(rows_per_w - 1, 0, (2 * rows_per_w - 2) % 4).wait()
        out_copy(rows_per_w - 1, 1, (2 * rows_per_w - 1) % 4).wait()

    return sc_kernel(word_padded, flat_idx)


def _tc_unpack_body(rows_ref, pos_ref, out_ref):
    x = rows_ref[...][:, :EMBED]
    x = x.reshape(TC_ROWS, SEQ, EMBED)
    out_ref[...] = x + pos_ref[...][None, :, :]


def _tc_unpack_acc_body(acc_ref, rows_ref, pos_ref, out_ref):
    del acc_ref
    _tc_unpack_body(rows_ref, pos_ref, out_ref)


def _tc_unpack(acc, rows128, pos_table, chunk, chunk_batch, batch):
    grid = chunk_batch // TC_ROWS

    def out_map(i, chunk=chunk, grid=grid):
        return (chunk * grid + i, 0, 0)

    out_shape = jax.ShapeDtypeStruct((batch, SEQ, EMBED), jnp.float32)
    data_specs = [
        pl.BlockSpec((TC_ROWS * SEQ, PAD), lambda i: (i, 0)),
        pl.BlockSpec((SEQ, EMBED), lambda i: (0, 0)),
    ]
    out_spec = pl.BlockSpec((TC_ROWS, SEQ, EMBED), out_map)
    if acc is None:
        return pl.pallas_call(
            _tc_unpack_body,
            out_shape=out_shape,
            grid=(grid,),
            in_specs=data_specs,
            out_specs=out_spec,
        )(rows128, pos_table)
    return pl.pallas_call(
        _tc_unpack_acc_body,
        out_shape=out_shape,
        grid=(grid,),
        in_specs=[pl.BlockSpec(memory_space=pltpu.MemorySpace.HBM)] + data_specs,
        out_specs=out_spec,
        input_output_aliases={0: 0},
    )(acc, rows128, pos_table)


def kernel(inputs, word_table, pos_table):
    batch, seq = inputs.shape
    chunk_batch = batch // NUM_CHUNKS
    chunk_idx = chunk_batch * seq
    flat_idx = inputs.reshape(batch * seq)
    word_padded = jnp.pad(word_table, ((0, 0), (0, PAD - EMBED)))

    acc = None
    for k in range(NUM_CHUNKS):
        rows128 = _sc_gather(
            word_padded,
            lax.slice_in_dim(flat_idx, k * chunk_idx, (k + 1) * chunk_idx),
            chunk_batch,
        )
        acc = _tc_unpack(acc, rows128, pos_table, k, chunk_batch, batch)
    return acc
